# baseline (speedup 1.0000x reference)
import functools

import jax
import jax.numpy as jnp
from jax import lax
from jax.experimental import pallas as pl
from jax.experimental.pallas import tpu as pltpu
from jax.experimental.pallas import tpu_sc as plsc

_NUM_CLASSES = 100000
_EMBED_DIM = 32
_MULT = 4
_BATCH = 16384

_NW = 32
_BSLOTS = _NW // _MULT
_B_PER_W = _BATCH // _BSLOTS
_CHUNK = 128
_N_CHUNKS = _B_PER_W // _CHUNK
_LANES = 16
_DT = _EMBED_DIM // 8


def _sc_embedding_lookup(feat_view, weight):
    mesh = plsc.VectorSubcoreMesh(core_axis_name="c", subcore_axis_name="s")

    @functools.partial(
        pl.kernel,
        mesh=mesh,
        out_type=jax.ShapeDtypeStruct(
            (_MULT, _DT, _BATCH // _CHUNK, 8, _CHUNK), jnp.float32),
        scratch_types=[
            pltpu.VMEM((_N_CHUNKS, 1, _CHUNK), jnp.int32),
            pltpu.VMEM((_N_CHUNKS, _CHUNK, _EMBED_DIM), jnp.float32),
            pltpu.VMEM((2, _DT, 1, 8, _CHUNK), jnp.float32),
            pltpu.SemaphoreType.DMA,
            pltpu.SemaphoreType.DMA,
        ],
        compiler_params=pltpu.CompilerParams(
            use_tc_tiling_on_sc=False, needs_layout_passes=False),
    )
    def body(feat_hbm, table_hbm, out_hbm, idx_v, rows_v, ttile_v, sem, osem):
        wid = lax.axis_index("s") * 2 + lax.axis_index("c")
        m = wid // _BSLOTS
        bt0 = (wid % _BSLOTS) * _N_CHUNKS
        pltpu.sync_copy(
            feat_hbm.at[pl.ds(bt0, _N_CHUNKS), pl.ds(m, 1), :], idx_v)

        off = jnp.full((_LANES,), _NUM_CLASSES, jnp.int32) * m
        iota = lax.iota(jnp.int32, _LANES)

        copies = []
        for j in range(_N_CHUNKS):
            def add_off(s, carry, j=j):
                sl = pl.ds(s * _LANES, _LANES)
                idx_v[j, 0, sl] = idx_v[j, 0, sl] + off
                return carry
            lax.fori_loop(0, _CHUNK // _LANES, add_off, 0, unroll=True)
            copies.append(
                pltpu.async_copy(
                    table_hbm.at[idx_v.at[j, 0]], rows_v.at[j], sem))

        out_copies = [None, None]
        for j in range(_N_CHUNKS):
            copies[j].wait()
            buf = j % 2
            if out_copies[buf] is not None:
                out_copies[buf].wait()

            def transpose_step(t, carry, j=j, buf=buf):
                d = t // 8
                qg = t % 8
                q_idx = qg * _LANES + iota
                d_idx = jnp.full((_LANES,), 1, jnp.int32) * d
                vals = plsc.load_gather(
                    rows_v, [jnp.full((_LANES,), j, jnp.int32), q_idx, d_idx])
                ttile_v[buf, d // 8, 0, d % 8, pl.ds(qg * _LANES, _LANES)] = vals
                return carry
            lax.fori_loop(0, _EMBED_DIM * (_CHUNK // _LANES),
                          transpose_step, 0, unroll=8)

            out_copies[buf] = pltpu.async_copy(
                ttile_v.at[buf], out_hbm.at[m, :, pl.ds(bt0 + j, 1)], osem)
        for oc in out_copies:
            if oc is not None:
                oc.wait()

    return body(feat_view, weight)


def kernel(feat, weight):
    feat_view = feat.reshape(_BATCH // _CHUNK, _CHUNK, _MULT).transpose(0, 2, 1)
    out5 = _sc_embedding_lookup(feat_view, weight)
    out = out5.transpose(2, 4, 0, 1, 3).reshape(_BATCH, _MULT, _EMBED_DIM)
    return out


# speedup vs baseline: 1.0047x; 1.0047x over previous
import functools

import jax
import jax.numpy as jnp
from jax import lax
from jax.experimental import pallas as pl
from jax.experimental.pallas import tpu as pltpu
from jax.experimental.pallas import tpu_sc as plsc

_NUM_CLASSES = 100000
_EMBED_DIM = 32
_MULT = 4
_BATCH = 16384

_NW = 32
_TOTAL = _BATCH * _MULT
_B_PER_W = _TOTAL // _NW
_CHUNK = 128
_N_CHUNKS = _B_PER_W // _CHUNK
_LANES = 16


def _sc_embedding_lookup(feat_grouped, weight):
    mesh = plsc.VectorSubcoreMesh(core_axis_name="c", subcore_axis_name="s")

    @functools.partial(
        pl.kernel,
        mesh=mesh,
        out_type=jax.ShapeDtypeStruct(
            (_NW, _N_CHUNKS, _CHUNK, _EMBED_DIM), jnp.float32),
        scratch_types=[
            pltpu.VMEM((_N_CHUNKS, _CHUNK), jnp.int32),
            pltpu.VMEM((_N_CHUNKS, _CHUNK, _EMBED_DIM), jnp.float32),
            pltpu.SemaphoreType.DMA,
        ],
        compiler_params=pltpu.CompilerParams(use_tc_tiling_on_sc=False),
    )
    def body(feat_hbm, table_hbm, out_hbm, idx_v, rows_v, sem):
        wid = lax.axis_index("s") * 2 + lax.axis_index("c")
        pltpu.sync_copy(feat_hbm.at[wid], idx_v)

        off = (lax.iota(jnp.int32, 16) % _MULT) * _NUM_CLASSES

        copies = []
        for j in range(_N_CHUNKS):
            def add_off(s, carry, j=j):
                sl = pl.ds(s * _LANES, _LANES)
                idx_v[j, sl] = idx_v[j, sl] + off
                return carry
            lax.fori_loop(0, _CHUNK // _LANES, add_off, 0, unroll=True)
            copies.append(
                pltpu.async_copy(table_hbm.at[idx_v.at[j]], rows_v.at[j], sem))
        for c in copies:
            c.wait()
        pltpu.sync_copy(rows_v, out_hbm.at[wid])

    return body(feat_grouped, weight)


def kernel(feat, weight):
    feat_grouped = feat.reshape(_NW, _N_CHUNKS, _CHUNK)
    out = _sc_embedding_lookup(feat_grouped, weight)
    return out.reshape(_BATCH, _MULT, _EMBED_DIM)
